# Initial kernel scaffold; baseline (speedup 1.0000x reference)
import jax
import jax.numpy as jnp
from jax.experimental import pallas as pl


def kernel(x, edge_index, W1l, b1l, W1r, W2l, b2l, W2r, gn_weight, gn_bias, gn_mean_scale):
    raise NotImplementedError("write your pallas kernel here")


# speedup vs baseline: 5.5758x; 5.5758x over previous
import functools

import jax
import jax.numpy as jnp
from jax import lax
from jax.experimental import pallas as pl
from jax.experimental.pallas import tpu as pltpu
from jax.experimental.pallas import tpu_sc as plsc

_N = 10000
_E = 320000
_D = 128
_NC = 2
_NS = 16
_NW = _NC * _NS
_EPT = _E // _NW
_K = 80
_NCHUNK = _EPT // _K
_NP = 10240
_RPT = _NP // _NS
_CW = 16
_ZB = 128


def _sc_segment_sum(h, src, dst, zrows):
  mesh = plsc.VectorSubcoreMesh(core_axis_name="c", subcore_axis_name="s",
                                num_cores=_NC, num_subcores=_NS)

  @functools.partial(
      pl.kernel,
      out_type=(jax.ShapeDtypeStruct((_NC, _NP, _D), jnp.float32),
                jax.ShapeDtypeStruct((_NW, _NP), jnp.float32)),
      mesh=mesh,
      scratch_types=[
          pltpu.VMEM_SHARED((_NP, _D), jnp.float32),
          pltpu.VMEM((_K,), jnp.int32),
          pltpu.VMEM((_K,), jnp.int32),
          pltpu.VMEM((_K, _D), jnp.float32),
          pltpu.VMEM((_NP,), jnp.float32),
          pltpu.VMEM((_ZB, _D), jnp.float32),
          pltpu.SemaphoreType.DMA,
      ],
      compiler_params=pltpu.CompilerParams(needs_layout_passes=False),
  )
  def seg_sum(h_hbm, src_hbm, dst_hbm, zrows_hbm,
              agg_out, cnt_out, agg_sp, idx_s, idx_d, rows, cnt_loc,
              buf, sem):
    c = lax.axis_index("c")
    s = lax.axis_index("s")
    wid = c * _NS + s
    r0 = s * _RPT
    pltpu.sync_copy(zrows_hbm, buf)
    for j in range(_RPT // _ZB):
      pltpu.sync_copy(buf, agg_sp.at[pl.ds(r0 + j * _ZB, _ZB)])
    zeros16 = jnp.zeros((16,), jnp.float32)

    def zi(i, carry):
      cnt_loc[pl.ds(i * 16, 16)] = zeros16
      return carry

    lax.fori_loop(0, _NP // 16, zi, 0)
    plsc.subcore_barrier()

    base = wid * _EPT
    ones16 = jnp.ones((16,), jnp.float32)

    def chunk(g, carry):
      off = base + g * _K
      pltpu.sync_copy(src_hbm.at[pl.ds(off, _K)], idx_s)
      pltpu.sync_copy(dst_hbm.at[pl.ds(off, _K)], idx_d)
      pltpu.async_copy(h_hbm.at[idx_s], rows, sem).wait()
      pltpu.sync_copy(rows, agg_sp.at[idx_d], add=True)
      for j in range(_K // 16):
        plsc.addupdate_scatter(cnt_loc, [idx_d[pl.ds(j * 16, 16)]], ones16)
      return carry

    lax.fori_loop(0, _NCHUNK, chunk, 0)
    plsc.subcore_barrier()

    for j in range(_RPT // _ZB):
      pltpu.sync_copy(agg_sp.at[pl.ds(r0 + j * _ZB, _ZB)], buf)
      pltpu.sync_copy(buf, agg_out.at[c, pl.ds(r0 + j * _ZB, _ZB)])
    pltpu.sync_copy(cnt_loc, cnt_out.at[wid])

  return seg_sum(h, src, dst, zrows)


def _tc_dense(p, cntp, h, Wl, bl, Wr, gw, gb, gms):
  def body(p_ref, cnt_ref, h_ref, wl_ref, bl_ref, wr_ref, gw_ref, gb_ref,
           gms_ref, out_ref):
    agg = p_ref[0, :_N] + p_ref[1, :_N]
    cnt_row = jnp.sum(cnt_ref[...], axis=0, keepdims=True)
    cnt = jnp.transpose(cnt_row[:, :_N])
    agg = agg / jnp.maximum(cnt, 1.0)
    z = (lax.dot_general(agg, wl_ref[...], (((1,), (1,)), ((), ())),
                         preferred_element_type=jnp.float32)
         + bl_ref[...]
         + lax.dot_general(h_ref[...], wr_ref[...], (((1,), (1,)), ((), ())),
                           preferred_element_type=jnp.float32))
    z = jnp.maximum(z, 0.0)
    mean = jnp.mean(z, axis=0, keepdims=True)
    out = z - mean * gms_ref[...]
    var = jnp.mean(out * out, axis=0, keepdims=True)
    out = out * lax.rsqrt(var + 1e-5)
    out_ref[...] = out * gw_ref[...] + gb_ref[...]

  return pl.pallas_call(
      body,
      out_shape=jax.ShapeDtypeStruct((_N, _D), jnp.float32),
  )(p, cntp, h, Wl, bl, Wr, gw, gb, gms)


def kernel(x, edge_index, W1l, b1l, W1r, W2l, b2l, W2r,
           gn_weight, gn_bias, gn_mean_scale):
  src = edge_index[0].astype(jnp.int32)
  dst = edge_index[1].astype(jnp.int32)
  zrows = jnp.zeros((_ZB, _D), jnp.float32)
  gw = gn_weight.reshape(1, _D)
  gb = gn_bias.reshape(1, _D)
  gms = gn_mean_scale.reshape(1, _D)
  h = x
  for Wl, bl, Wr in ((W1l, b1l, W1r), (W2l, b2l, W2r)):
    p, cntp = _sc_segment_sum(h, src, dst, zrows)
    h = _tc_dense(p, cntp, h, Wl, bl.reshape(1, _D), Wr, gw, gb, gms)
  return h


# speedup vs baseline: 12.6742x; 2.2731x over previous
import functools

import jax
import jax.numpy as jnp
from jax import lax
from jax.experimental import pallas as pl
from jax.experimental.pallas import tpu as pltpu
from jax.experimental.pallas import tpu_sc as plsc

_N = 10000
_E = 320000
_D = 128
_NC = 2
_NS = 16
_NW = _NC * _NS
_EPT = _E // _NW
_K = 80
_NCHUNK = _EPT // _K
_NP = 10240
_RPT = _NP // _NS


def _sc_segment_sum(h, src, dst, zrows):
  mesh = plsc.VectorSubcoreMesh(core_axis_name="c", subcore_axis_name="s",
                                num_cores=_NC, num_subcores=_NS)

  @functools.partial(
      pl.kernel,
      out_type=(jax.ShapeDtypeStruct((_NC, _NP, _D), jnp.float32),
                jax.ShapeDtypeStruct((_NW, _NP), jnp.float32)),
      mesh=mesh,
      scratch_types=[
          pltpu.VMEM_SHARED((_NP, _D), jnp.float32),
          pltpu.VMEM((_EPT,), jnp.int32),
          pltpu.VMEM((_K,), jnp.int32),
          pltpu.VMEM((_K,), jnp.int32),
          pltpu.VMEM((_K, _D), jnp.float32),
          pltpu.VMEM((_K, _D), jnp.float32),
          pltpu.VMEM((_NP,), jnp.float32),
          pltpu.SemaphoreType.DMA,
          pltpu.SemaphoreType.DMA,
          pltpu.SemaphoreType.DMA,
          pltpu.SemaphoreType.DMA,
      ],
      compiler_params=pltpu.CompilerParams(needs_layout_passes=False),
  )
  def seg_sum(h_hbm, src_hbm, dst_hbm, zrows_hbm,
              agg_out, cnt_out, agg_sp, idx_all_s,
              idx_d_a, idx_d_b, rows_a, rows_b, cnt_loc,
              sem_ra, sem_rb, sem_ia, sem_ib):
    c = lax.axis_index("c")
    s = lax.axis_index("s")
    wid = c * _NS + s
    r0 = s * _RPT
    base = wid * _EPT
    pltpu.sync_copy(src_hbm.at[pl.ds(base, _EPT)], idx_all_s)
    pltpu.sync_copy(zrows_hbm, rows_a)
    for j in range(_RPT // _K):
      pltpu.sync_copy(rows_a, agg_sp.at[pl.ds(r0 + j * _K, _K)])
    zeros16 = jnp.zeros((16,), jnp.float32)

    def zi(i, carry):
      cnt_loc[pl.ds(i * 16, 16)] = zeros16
      return carry

    lax.fori_loop(0, _NP // 16, zi, 0)
    plsc.subcore_barrier()

    ones16 = jnp.ones((16,), jnp.float32)

    def fetch(g, idx_d_buf, rows_buf, sem_r, sem_i):
      pltpu.async_copy(dst_hbm.at[pl.ds(base + g * _K, _K)], idx_d_buf,
                       sem_i)
      pltpu.async_copy(h_hbm.at[idx_all_s.at[pl.ds(g * _K, _K)]],
                       rows_buf, sem_r)

    def wait_chunk(idx_d_buf, rows_buf, sem_r, sem_i):
      pltpu.make_async_copy(dst_hbm.at[pl.ds(0, _K)], idx_d_buf,
                            sem_i).wait()
      pltpu.make_async_copy(h_hbm.at[pl.ds(0, _K)], rows_buf, sem_r).wait()

    def hist(idx_d_buf):
      for j in range(_K // 16):
        plsc.addupdate_scatter(cnt_loc, [idx_d_buf[pl.ds(j * 16, 16)]],
                               ones16)

    fetch(0, idx_d_a, rows_a, sem_ra, sem_ia)
    fetch(1, idx_d_b, rows_b, sem_rb, sem_ib)

    def body(t, carry):
      for bi, (idx_d_buf, rows_buf, sem_r, sem_i) in enumerate(
          ((idx_d_a, rows_a, sem_ra, sem_ia),
           (idx_d_b, rows_b, sem_rb, sem_ib))):
        g = 2 * t + bi
        wait_chunk(idx_d_buf, rows_buf, sem_r, sem_i)
        pltpu.sync_copy(rows_buf, agg_sp.at[idx_d_buf], add=True)
        hist(idx_d_buf)
        pg = g + 2

        @pl.when(pg < _NCHUNK)
        def _():
          fetch(pg, idx_d_buf, rows_buf, sem_r, sem_i)

      return carry

    lax.fori_loop(0, _NCHUNK // 2, body, 0)
    wait_chunk(idx_d_a, rows_a, sem_ra, sem_ia)
    pltpu.sync_copy(rows_a, agg_sp.at[idx_d_a], add=True)
    hist(idx_d_a)
    plsc.subcore_barrier()

    for j in range(_RPT // _K):
      pltpu.sync_copy(agg_sp.at[pl.ds(r0 + j * _K, _K)], rows_a)
      pltpu.sync_copy(rows_a, agg_out.at[c, pl.ds(r0 + j * _K, _K)])
    pltpu.sync_copy(cnt_loc, cnt_out.at[wid])

  return seg_sum(h, src, dst, zrows)


def _tc_dense(p, cntp, h, Wl, bl, Wr, gw, gb, gms):
  def body(p_ref, cnt_ref, h_ref, wl_ref, bl_ref, wr_ref, gw_ref, gb_ref,
           gms_ref, out_ref):
    agg = p_ref[0, :_N] + p_ref[1, :_N]
    cnt_row = jnp.sum(cnt_ref[...], axis=0, keepdims=True)
    cnt = jnp.transpose(cnt_row[:, :_N])
    agg = agg / jnp.maximum(cnt, 1.0)
    z = (lax.dot_general(agg, wl_ref[...], (((1,), (1,)), ((), ())),
                         preferred_element_type=jnp.float32)
         + bl_ref[...]
         + lax.dot_general(h_ref[...], wr_ref[...], (((1,), (1,)), ((), ())),
                           preferred_element_type=jnp.float32))
    z = jnp.maximum(z, 0.0)
    mean = jnp.mean(z, axis=0, keepdims=True)
    out = z - mean * gms_ref[...]
    var = jnp.mean(out * out, axis=0, keepdims=True)
    out = out * lax.rsqrt(var + 1e-5)
    out_ref[...] = out * gw_ref[...] + gb_ref[...]

  return pl.pallas_call(
      body,
      out_shape=jax.ShapeDtypeStruct((_N, _D), jnp.float32),
  )(p, cntp, h, Wl, bl, Wr, gw, gb, gms)


def kernel(x, edge_index, W1l, b1l, W1r, W2l, b2l, W2r,
           gn_weight, gn_bias, gn_mean_scale):
  src = edge_index[0].astype(jnp.int32)
  dst = edge_index[1].astype(jnp.int32)
  zrows = jnp.zeros((_K, _D), jnp.float32)
  gw = gn_weight.reshape(1, _D)
  gb = gn_bias.reshape(1, _D)
  gms = gn_mean_scale.reshape(1, _D)
  h = x
  for Wl, bl, Wr in ((W1l, b1l, W1r), (W2l, b2l, W2r)):
    p, cntp = _sc_segment_sum(h, src, dst, zrows)
    h = _tc_dense(p, cntp, h, Wl, bl.reshape(1, _D), Wr, gw, gb, gms)
  return h
